# baseline (speedup 1.0000x reference)
import math

import jax
import jax.numpy as jnp
from jax import lax
from jax.experimental import pallas as pl
from jax.experimental.pallas import tpu as pltpu
from jax.experimental.pallas import tpu_sc as plsc

L = 16
T = 524288
TMASK = T - 1
F = 2
N_MIN = 16
N_MAX = 2048
N_POINTS = 262144
PRIME = 131101

_B = math.exp((math.log(N_MAX) - math.log(N_MIN)) / (L - 1))
RES = []
HASHED = []
OFFS = []
_off = 0
for _i in range(L):
    _res = int(N_MIN * _B ** _i)
    RES.append(_res)
    _n = (_res + 1) ** 2
    _h = _n > T
    HASHED.append(_h)
    OFFS.append(_off)
    _off += T if _h else _n
TOTAL = _off

NC = 2
NS = 16
NW = NC * NS
LANES = 16
PPW = N_POINTS // NW
C = 1024
NCHUNK = PPW // C
NG = C // LANES

NLVL_VMEM = 8
NLVL_SPMEM = 12
TAB_ROWS = OFFS[NLVL_VMEM]
SP_START = OFFS[NLVL_VMEM] & ~7
SP_ROWS = (OFFS[NLVL_SPMEM] - SP_START + 7) & ~7
HMASK = -65536


def _body(uvf_hbm, lat_hbm, out_hbm, uv_v, u_v, v_v, tab_v, sp_tab,
          idx_v, w_v, f_v, out_v, sems):
    sid = lax.axis_index("s")
    wid = sid * NC + lax.axis_index("c")
    base = wid * PPW
    iota = lax.iota(jnp.int32, LANES)

    pltpu.sync_copy(lat_hbm.at[pl.ds(0, TAB_ROWS)], tab_v)

    @pl.when(sid == 0)
    def _stage():
        pltpu.sync_copy(lat_hbm.at[pl.ds(SP_START, SP_ROWS)], sp_tab)

    plsc.subcore_barrier()

    def unpack(g):
        g0 = plsc.bitcast(g << 16, jnp.float32)
        g1 = plsc.bitcast(g & HMASK, jnp.float32)
        return g0, g1

    def corners(g, res, off, hashed):
        s = g * LANES
        uu = u_v[pl.ds(s, LANES)] * float(res)
        vv = v_v[pl.ds(s, LANES)] * float(res)
        iu = uu.astype(jnp.int32)
        iv = vv.astype(jnp.int32)
        fu = uu - iu.astype(jnp.float32)
        fv = vv - iv.astype(jnp.float32)
        if hashed:
            a0 = iv * PRIME
            a1 = a0 + PRIME
            i00 = ((iu ^ a0) & TMASK) + off
            i01 = ((iu ^ a1) & TMASK) + off
            iu1 = iu + 1
            i10 = ((iu1 ^ a0) & TMASK) + off
            i11 = ((iu1 ^ a1) & TMASK) + off
        else:
            b0 = iu * res + iv + off
            i00 = b0
            i01 = b0 + 1
            i10 = b0 + res
            i11 = b0 + res + 1
        gu = 1.0 - fu
        gv = 1.0 - fv
        return (i00, i01, i10, i11), (gu * gv, gu * fv, fu * gv, fu * fv)

    def chunk_body(ci, _):
        pt0 = base + ci * C
        pltpu.sync_copy(uvf_hbm.at[pl.ds(2 * pt0, 2 * C)], uv_v)

        def deint(g, _):
            j2 = 2 * (g * LANES + iota)
            u_v[pl.ds(g * LANES, LANES)] = plsc.load_gather(uv_v, [j2])
            v_v[pl.ds(g * LANES, LANES)] = plsc.load_gather(uv_v, [j2 + 1])
            return 0

        lax.fori_loop(0, NG, deint, 0)

        def pass_a(lvl, par):
            res, off, hashed = RES[lvl], OFFS[lvl], HASHED[lvl]
            shift = SP_START if lvl < NLVL_SPMEM else 0

            def work(g, _, res=res, off=off, hashed=hashed, shift=shift):
                idx, w = corners(g, res, off, hashed)
                for k in range(4):
                    p = k * C + g * LANES
                    idx_v[par, pl.ds(p, LANES)] = idx[k] - shift
                    w_v[par, pl.ds(p, LANES)] = w[k]
                return 0

            lax.fori_loop(0, NG, work, 0)

        def fire(lvl, par):
            src = sp_tab if lvl < NLVL_SPMEM else lat_hbm
            return pltpu.async_copy(src.at[idx_v.at[par]], f_v.at[par],
                                    sems.at[par])

        def pass_b(lvl, par):
            def work(g, _, lvl=lvl, par=par):
                s = g * LANES
                acc0 = jnp.zeros((LANES,), jnp.float32)
                acc1 = jnp.zeros((LANES,), jnp.float32)
                for k in range(4):
                    p = k * C + s
                    w = w_v[par, pl.ds(p, LANES)]
                    g0, g1 = unpack(f_v[par, pl.ds(p, LANES)])
                    acc0 = acc0 + g0 * w
                    acc1 = acc1 + g1 * w
                pos = (s + iota) * (L * F) + (2 * lvl)
                plsc.store_scatter(out_v, [pos], acc0)
                plsc.store_scatter(out_v, [pos + 1], acc1)
                return 0

            lax.fori_loop(0, NG, work, 0)

        def vmem_level(lvl):
            res, off = RES[lvl], OFFS[lvl]

            def work(g, _, lvl=lvl, res=res, off=off):
                s = g * LANES
                idx, w = corners(g, res, off, False)
                acc0 = jnp.zeros((LANES,), jnp.float32)
                acc1 = jnp.zeros((LANES,), jnp.float32)
                for k in range(4):
                    g0, g1 = unpack(plsc.load_gather(tab_v, [idx[k]]))
                    acc0 = acc0 + g0 * w[k]
                    acc1 = acc1 + g1 * w[k]
                pos = (s + iota) * (L * F) + (2 * lvl)
                plsc.store_scatter(out_v, [pos], acc0)
                plsc.store_scatter(out_v, [pos + 1], acc1)
                return 0

            lax.fori_loop(0, NG, work, 0)

        pass_a(NLVL_VMEM, NLVL_VMEM % 2)
        cps = {NLVL_VMEM: fire(NLVL_VMEM, NLVL_VMEM % 2)}
        for lvl in range(NLVL_VMEM):
            vmem_level(lvl)
        for lvl in range(NLVL_VMEM, L):
            if lvl + 1 < L:
                par = (lvl + 1) % 2
                pass_a(lvl + 1, par)
                cps[lvl + 1] = fire(lvl + 1, par)
            cps.pop(lvl).wait()
            pass_b(lvl, lvl % 2)

        pltpu.sync_copy(out_v, out_hbm.at[pl.ds(pt0 * (L * F), C * L * F)])
        return 0

    lax.fori_loop(0, NCHUNK, chunk_body, 0)


@jax.jit
def kernel(uv, latents):
    uvf = uv.reshape(-1)
    packed = jax.lax.bitcast_convert_type(
        latents.astype(jnp.bfloat16), jnp.int32)
    mesh = plsc.VectorSubcoreMesh(
        core_axis_name="c", subcore_axis_name="s",
        num_cores=NC, num_subcores=NS)
    run = pl.kernel(
        _body,
        out_type=jax.ShapeDtypeStruct((N_POINTS * L * F,), jnp.float32),
        mesh=mesh,
        compiler_params=pltpu.CompilerParams(
            needs_layout_passes=False, use_tc_tiling_on_sc=False),
        scratch_types=[
            pltpu.VMEM((2 * C,), jnp.float32),
            pltpu.VMEM((C,), jnp.float32),
            pltpu.VMEM((C,), jnp.float32),
            pltpu.VMEM((TAB_ROWS,), jnp.int32),
            pltpu.VMEM_SHARED((SP_ROWS,), jnp.int32),
            pltpu.VMEM((2, 4 * C), jnp.int32),
            pltpu.VMEM((2, 4 * C), jnp.float32),
            pltpu.VMEM((2, 4 * C), jnp.int32),
            pltpu.VMEM((C * L * F,), jnp.float32),
            pltpu.SemaphoreType.DMA((2,)),
        ],
    )
    return run(uvf, packed).reshape(N_POINTS, L * F)


# speedup vs baseline: 1.2458x; 1.2458x over previous
import math

import jax
import jax.numpy as jnp
from jax import lax
from jax.experimental import pallas as pl
from jax.experimental.pallas import tpu as pltpu
from jax.experimental.pallas import tpu_sc as plsc

L = 16
T = 524288
TMASK = T - 1
F = 2
N_MIN = 16
N_MAX = 2048
N_POINTS = 262144
PRIME = 131101

_B = math.exp((math.log(N_MAX) - math.log(N_MIN)) / (L - 1))
RES = []
HASHED = []
OFFS = []
_off = 0
for _i in range(L):
    _res = int(N_MIN * _B ** _i)
    RES.append(_res)
    _n = (_res + 1) ** 2
    _h = _n > T
    HASHED.append(_h)
    OFFS.append(_off)
    _off += T if _h else _n
TOTAL = _off

NC = 2
NS = 16
NW = NC * NS
LANES = 16
PPW = N_POINTS // NW
C = 1024
NCHUNK = PPW // C
NG = C // LANES

NLVL_VMEM = 7
NLVL_SPMEM = 12
TAB_ROWS = OFFS[NLVL_VMEM]
SP_START = OFFS[NLVL_VMEM] & ~7
SP_ROWS = (OFFS[NLVL_SPMEM] - SP_START + 7) & ~7
HMASK = -65536


def _body(u_hbm, v_hbm, lat_hbm, out_hbm, u_v, v_v, tab_v, sp_tab,
          idx_v, w_v, f_v, out_v, sems):
    sid = lax.axis_index("s")
    wid = sid * NC + lax.axis_index("c")
    base = wid * PPW
    iota = lax.iota(jnp.int32, LANES)

    pltpu.sync_copy(lat_hbm.at[pl.ds(0, TAB_ROWS)], tab_v)

    @pl.when(sid == 0)
    def _stage():
        pltpu.sync_copy(lat_hbm.at[pl.ds(SP_START, SP_ROWS)], sp_tab)

    plsc.subcore_barrier()

    def unpack(g):
        g0 = plsc.bitcast(g << 16, jnp.float32)
        g1 = plsc.bitcast(g & HMASK, jnp.float32)
        return g0, g1

    def corners(g, res, off, hashed):
        s = g * LANES
        uu = u_v[pl.ds(s, LANES)] * float(res)
        vv = v_v[pl.ds(s, LANES)] * float(res)
        iu = uu.astype(jnp.int32)
        iv = vv.astype(jnp.int32)
        fu = uu - iu.astype(jnp.float32)
        fv = vv - iv.astype(jnp.float32)
        if hashed:
            a0 = iv * PRIME
            a1 = a0 + PRIME
            i00 = ((iu ^ a0) & TMASK) + off
            i01 = ((iu ^ a1) & TMASK) + off
            iu1 = iu + 1
            i10 = ((iu1 ^ a0) & TMASK) + off
            i11 = ((iu1 ^ a1) & TMASK) + off
        else:
            b0 = iu * res + iv + off
            i00 = b0
            i01 = b0 + 1
            i10 = b0 + res
            i11 = b0 + res + 1
        gu = 1.0 - fu
        gv = 1.0 - fv
        return (i00, i01, i10, i11), (gu * gv, gu * fv, fu * gv, fu * fv)

    def chunk_body(ci, _):
        pt0 = base + ci * C
        pltpu.sync_copy(u_hbm.at[pl.ds(pt0, C)], u_v)
        pltpu.sync_copy(v_hbm.at[pl.ds(pt0, C)], v_v)

        def pass_a(lvl, par):
            res, off, hashed = RES[lvl], OFFS[lvl], HASHED[lvl]
            shift = SP_START if lvl < NLVL_SPMEM else 0

            def work(g, _, res=res, off=off, hashed=hashed, shift=shift):
                idx, w = corners(g, res, off, hashed)
                for k in range(4):
                    p = k * C + g * LANES
                    idx_v[par, pl.ds(p, LANES)] = idx[k] - shift
                    w_v[par, pl.ds(p, LANES)] = w[k]
                return 0

            lax.fori_loop(0, NG, work, 0)

        def fire(lvl, par):
            src = sp_tab if lvl < NLVL_SPMEM else lat_hbm
            return pltpu.async_copy(src.at[idx_v.at[par]], f_v.at[par],
                                    sems.at[par])

        def pass_b(lvl, par):
            def work(g, _, lvl=lvl, par=par):
                s = g * LANES
                acc0 = jnp.zeros((LANES,), jnp.float32)
                acc1 = jnp.zeros((LANES,), jnp.float32)
                for k in range(4):
                    p = k * C + s
                    w = w_v[par, pl.ds(p, LANES)]
                    g0, g1 = unpack(f_v[par, pl.ds(p, LANES)])
                    acc0 = acc0 + g0 * w
                    acc1 = acc1 + g1 * w
                pos = (s + iota) * (L * F) + (2 * lvl)
                plsc.store_scatter(out_v, [pos], acc0)
                plsc.store_scatter(out_v, [pos + 1], acc1)
                return 0

            lax.fori_loop(0, NG, work, 0)

        def vmem_level(lvl):
            res, off = RES[lvl], OFFS[lvl]

            def work(g, _, lvl=lvl, res=res, off=off):
                s = g * LANES
                idx, w = corners(g, res, off, False)
                acc0 = jnp.zeros((LANES,), jnp.float32)
                acc1 = jnp.zeros((LANES,), jnp.float32)
                for k in range(4):
                    g0, g1 = unpack(plsc.load_gather(tab_v, [idx[k]]))
                    acc0 = acc0 + g0 * w[k]
                    acc1 = acc1 + g1 * w[k]
                pos = (s + iota) * (L * F) + (2 * lvl)
                plsc.store_scatter(out_v, [pos], acc0)
                plsc.store_scatter(out_v, [pos + 1], acc1)
                return 0

            lax.fori_loop(0, NG, work, 0)

        pass_a(NLVL_VMEM, NLVL_VMEM % 2)
        cps = {NLVL_VMEM: fire(NLVL_VMEM, NLVL_VMEM % 2)}
        for lvl in range(NLVL_VMEM):
            vmem_level(lvl)
        for lvl in range(NLVL_VMEM, L):
            if lvl + 1 < L:
                par = (lvl + 1) % 2
                pass_a(lvl + 1, par)
                cps[lvl + 1] = fire(lvl + 1, par)
            cps.pop(lvl).wait()
            pass_b(lvl, lvl % 2)

        pltpu.sync_copy(out_v, out_hbm.at[pl.ds(pt0 * (L * F), C * L * F)])
        return 0

    lax.fori_loop(0, NCHUNK, chunk_body, 0)


@jax.jit
def kernel(uv, latents):
    u = uv[:, 0]
    v = uv[:, 1]
    packed = jax.lax.bitcast_convert_type(
        latents.astype(jnp.bfloat16), jnp.int32)
    mesh = plsc.VectorSubcoreMesh(
        core_axis_name="c", subcore_axis_name="s",
        num_cores=NC, num_subcores=NS)
    run = pl.kernel(
        _body,
        out_type=jax.ShapeDtypeStruct((N_POINTS * L * F,), jnp.float32),
        mesh=mesh,
        compiler_params=pltpu.CompilerParams(
            needs_layout_passes=False, use_tc_tiling_on_sc=False),
        scratch_types=[
            pltpu.VMEM((C,), jnp.float32),
            pltpu.VMEM((C,), jnp.float32),
            pltpu.VMEM((TAB_ROWS,), jnp.int32),
            pltpu.VMEM_SHARED((SP_ROWS,), jnp.int32),
            pltpu.VMEM((2, 4 * C), jnp.int32),
            pltpu.VMEM((2, 4 * C), jnp.float32),
            pltpu.VMEM((2, 4 * C), jnp.int32),
            pltpu.VMEM((C * L * F,), jnp.float32),
            pltpu.SemaphoreType.DMA((2,)),
        ],
    )
    return run(u, v, packed).reshape(N_POINTS, L * F)


# speedup vs baseline: 1.3170x; 1.0571x over previous
import math

import jax
import jax.numpy as jnp
from jax import lax
from jax.experimental import pallas as pl
from jax.experimental.pallas import tpu as pltpu
from jax.experimental.pallas import tpu_sc as plsc

L = 16
T = 524288
TMASK = T - 1
F = 2
N_MIN = 16
N_MAX = 2048
N_POINTS = 262144
PRIME = 131101

_B = math.exp((math.log(N_MAX) - math.log(N_MIN)) / (L - 1))
RES = []
HASHED = []
OFFS = []
_off = 0
for _i in range(L):
    _res = int(N_MIN * _B ** _i)
    RES.append(_res)
    _n = (_res + 1) ** 2
    _h = _n > T
    HASHED.append(_h)
    OFFS.append(_off)
    _off += T if _h else _n
TOTAL = _off

NC = 2
NS = 16
NW = NC * NS
LANES = 16
PPW = N_POINTS // NW
C = 1024
NCHUNK = PPW // C
NG = C // LANES

NLVL_VMEM = 7
NLVL_SPMEM = 11
TAB_ROWS = OFFS[NLVL_VMEM]
SP_START = OFFS[NLVL_VMEM] & ~7
SP_ROWS = (OFFS[NLVL_SPMEM] - SP_START + 7) & ~7
HMASK = -65536


def _body(u_hbm, v_hbm, lat_hbm, out_hbm, u_v, v_v, tab_v, sp_tab,
          idx_v, w_v, f_v, out_v, sems):
    sid = lax.axis_index("s")
    wid = sid * NC + lax.axis_index("c")
    base = wid * PPW
    iota = lax.iota(jnp.int32, LANES)

    pltpu.sync_copy(lat_hbm.at[pl.ds(0, TAB_ROWS)], tab_v)

    @pl.when(sid == 0)
    def _stage():
        pltpu.sync_copy(lat_hbm.at[pl.ds(SP_START, SP_ROWS)], sp_tab)

    plsc.subcore_barrier()

    def unpack(g):
        g0 = plsc.bitcast(g << 16, jnp.float32)
        g1 = plsc.bitcast(g & HMASK, jnp.float32)
        return g0, g1

    def corners(g, res, off, hashed):
        s = g * LANES
        uu = u_v[pl.ds(s, LANES)] * float(res)
        vv = v_v[pl.ds(s, LANES)] * float(res)
        iu = uu.astype(jnp.int32)
        iv = vv.astype(jnp.int32)
        fu = uu - iu.astype(jnp.float32)
        fv = vv - iv.astype(jnp.float32)
        if hashed:
            a0 = iv * PRIME
            a1 = a0 + PRIME
            i00 = ((iu ^ a0) & TMASK) + off
            i01 = ((iu ^ a1) & TMASK) + off
            iu1 = iu + 1
            i10 = ((iu1 ^ a0) & TMASK) + off
            i11 = ((iu1 ^ a1) & TMASK) + off
        else:
            b0 = iu * res + iv + off
            i00 = b0
            i01 = b0 + 1
            i10 = b0 + res
            i11 = b0 + res + 1
        gu = 1.0 - fu
        gv = 1.0 - fv
        return (i00, i01, i10, i11), (gu * gv, gu * fv, fu * gv, fu * fv)

    def chunk_body(ci, _):
        pt0 = base + ci * C
        pltpu.sync_copy(u_hbm.at[pl.ds(pt0, C)], u_v)
        pltpu.sync_copy(v_hbm.at[pl.ds(pt0, C)], v_v)

        def pass_a(lvl, par):
            res, off, hashed = RES[lvl], OFFS[lvl], HASHED[lvl]
            shift = SP_START if lvl < NLVL_SPMEM else 0

            def work(g, res=res, off=off, hashed=hashed, shift=shift):
                idx, w = corners(g, res, off, hashed)
                for k in range(4):
                    p = k * C + g * LANES
                    idx_v[par, pl.ds(p, LANES)] = idx[k] - shift
                    w_v[par, pl.ds(p, LANES)] = w[k]

            plsc.parallel_loop(0, NG, unroll=2)(work)

        def fire(lvl, par):
            src = sp_tab if lvl < NLVL_SPMEM else lat_hbm
            return pltpu.async_copy(src.at[idx_v.at[par]], f_v.at[par],
                                    sems.at[par])

        def pass_b(lvl, par):
            def work(g, lvl=lvl, par=par):
                s = g * LANES
                acc0 = jnp.zeros((LANES,), jnp.float32)
                acc1 = jnp.zeros((LANES,), jnp.float32)
                for k in range(4):
                    p = k * C + s
                    w = w_v[par, pl.ds(p, LANES)]
                    g0, g1 = unpack(f_v[par, pl.ds(p, LANES)])
                    acc0 = acc0 + g0 * w
                    acc1 = acc1 + g1 * w
                pos = (s + iota) * (L * F) + (2 * lvl)
                plsc.store_scatter(out_v, [pos], acc0)
                plsc.store_scatter(out_v, [pos + 1], acc1)

            plsc.parallel_loop(0, NG, unroll=2)(work)

        def vmem_level(lvl):
            res, off = RES[lvl], OFFS[lvl]

            def work(g, lvl=lvl, res=res, off=off):
                s = g * LANES
                idx, w = corners(g, res, off, False)
                acc0 = jnp.zeros((LANES,), jnp.float32)
                acc1 = jnp.zeros((LANES,), jnp.float32)
                for k in range(4):
                    g0, g1 = unpack(plsc.load_gather(tab_v, [idx[k]]))
                    acc0 = acc0 + g0 * w[k]
                    acc1 = acc1 + g1 * w[k]
                pos = (s + iota) * (L * F) + (2 * lvl)
                plsc.store_scatter(out_v, [pos], acc0)
                plsc.store_scatter(out_v, [pos + 1], acc1)

            plsc.parallel_loop(0, NG, unroll=2)(work)

        pass_a(NLVL_VMEM, NLVL_VMEM % 2)
        cps = {NLVL_VMEM: fire(NLVL_VMEM, NLVL_VMEM % 2)}
        for lvl in range(NLVL_VMEM):
            vmem_level(lvl)
        for lvl in range(NLVL_VMEM, L):
            if lvl + 1 < L:
                par = (lvl + 1) % 2
                pass_a(lvl + 1, par)
                cps[lvl + 1] = fire(lvl + 1, par)
            cps.pop(lvl).wait()
            pass_b(lvl, lvl % 2)

        pltpu.sync_copy(out_v, out_hbm.at[pl.ds(pt0 * (L * F), C * L * F)])
        return 0

    lax.fori_loop(0, NCHUNK, chunk_body, 0)


@jax.jit
def kernel(uv, latents):
    u = uv[:, 0]
    v = uv[:, 1]
    packed = jax.lax.bitcast_convert_type(
        latents.astype(jnp.bfloat16), jnp.int32)
    mesh = plsc.VectorSubcoreMesh(
        core_axis_name="c", subcore_axis_name="s",
        num_cores=NC, num_subcores=NS)
    run = pl.kernel(
        _body,
        out_type=jax.ShapeDtypeStruct((N_POINTS * L * F,), jnp.float32),
        mesh=mesh,
        compiler_params=pltpu.CompilerParams(
            needs_layout_passes=False, use_tc_tiling_on_sc=False),
        scratch_types=[
            pltpu.VMEM((C,), jnp.float32),
            pltpu.VMEM((C,), jnp.float32),
            pltpu.VMEM((TAB_ROWS,), jnp.int32),
            pltpu.VMEM_SHARED((SP_ROWS,), jnp.int32),
            pltpu.VMEM((2, 4 * C), jnp.int32),
            pltpu.VMEM((2, 4 * C), jnp.float32),
            pltpu.VMEM((2, 4 * C), jnp.int32),
            pltpu.VMEM((C * L * F,), jnp.float32),
            pltpu.SemaphoreType.DMA((2,)),
        ],
    )
    return run(u, v, packed).reshape(N_POINTS, L * F)


# speedup vs baseline: 1.3825x; 1.0498x over previous
import math

import jax
import jax.numpy as jnp
from jax import lax
from jax.experimental import pallas as pl
from jax.experimental.pallas import tpu as pltpu
from jax.experimental.pallas import tpu_sc as plsc

L = 16
T = 524288
TMASK = T - 1
F = 2
N_MIN = 16
N_MAX = 2048
N_POINTS = 262144
PRIME = 131101

_B = math.exp((math.log(N_MAX) - math.log(N_MIN)) / (L - 1))
RES = []
HASHED = []
OFFS = []
_off = 0
for _i in range(L):
    _res = int(N_MIN * _B ** _i)
    RES.append(_res)
    _n = (_res + 1) ** 2
    _h = _n > T
    HASHED.append(_h)
    OFFS.append(_off)
    _off += T if _h else _n
TOTAL = _off

NC = 2
NS = 16
NW = NC * NS
LANES = 16
PPW = N_POINTS // NW
C = 1024
NCHUNK = PPW // C
NG = C // LANES

NLVL_VMEM = 7
NLVL_SPMEM = 12
TAB_ROWS = OFFS[NLVL_VMEM]
SP_START = OFFS[NLVL_VMEM] & ~7
SP_ROWS = (OFFS[NLVL_SPMEM] - SP_START + 7) & ~7
HMASK = -65536


def _body(u_hbm, v_hbm, lat_hbm, out_hbm, u_v, v_v, tab_v, sp_tab,
          idx_v, w_v, f_v, out_v, sems):
    sid = lax.axis_index("s")
    wid = sid * NC + lax.axis_index("c")
    base = wid * PPW
    iota = lax.iota(jnp.int32, LANES)

    pltpu.sync_copy(lat_hbm.at[pl.ds(0, TAB_ROWS)], tab_v)

    @pl.when(sid == 0)
    def _stage():
        pltpu.sync_copy(lat_hbm.at[pl.ds(SP_START, SP_ROWS)], sp_tab)

    plsc.subcore_barrier()

    def unpack(g):
        g0 = plsc.bitcast(g << 16, jnp.float32)
        g1 = plsc.bitcast(g & HMASK, jnp.float32)
        return g0, g1

    def corners(g, res, off, hashed):
        s = g * LANES
        uu = u_v[pl.ds(s, LANES)] * float(res)
        vv = v_v[pl.ds(s, LANES)] * float(res)
        iu = uu.astype(jnp.int32)
        iv = vv.astype(jnp.int32)
        fu = uu - iu.astype(jnp.float32)
        fv = vv - iv.astype(jnp.float32)
        if hashed:
            a0 = iv * PRIME
            a1 = a0 + PRIME
            i00 = ((iu ^ a0) & TMASK) + off
            i01 = ((iu ^ a1) & TMASK) + off
            iu1 = iu + 1
            i10 = ((iu1 ^ a0) & TMASK) + off
            i11 = ((iu1 ^ a1) & TMASK) + off
        else:
            b0 = iu * res + iv + off
            i00 = b0
            i01 = b0 + 1
            i10 = b0 + res
            i11 = b0 + res + 1
        gu = 1.0 - fu
        gv = 1.0 - fv
        return (i00, i01, i10, i11), (gu * gv, gu * fv, fu * gv, fu * fv)

    def chunk_body(ci, _):
        pt0 = base + ci * C
        pltpu.sync_copy(u_hbm.at[pl.ds(pt0, C)], u_v)
        pltpu.sync_copy(v_hbm.at[pl.ds(pt0, C)], v_v)

        def pass_a(lvl, par):
            res, off, hashed = RES[lvl], OFFS[lvl], HASHED[lvl]
            shift = SP_START if lvl < NLVL_SPMEM else 0

            def work(g, res=res, off=off, hashed=hashed, shift=shift):
                idx, w = corners(g, res, off, hashed)
                for k in range(4):
                    p = k * C + g * LANES
                    idx_v[par, pl.ds(p, LANES)] = idx[k] - shift
                    w_v[par, pl.ds(p, LANES)] = w[k]

            plsc.parallel_loop(0, NG, unroll=2)(work)

        def fire(lvl, par):
            src = sp_tab if lvl < NLVL_SPMEM else lat_hbm
            return pltpu.async_copy(src.at[idx_v.at[par]], f_v.at[par],
                                    sems.at[par])

        def pass_b(lvl, par):
            def work(g, lvl=lvl, par=par):
                s = g * LANES
                acc0 = jnp.zeros((LANES,), jnp.float32)
                acc1 = jnp.zeros((LANES,), jnp.float32)
                for k in range(4):
                    p = k * C + s
                    w = w_v[par, pl.ds(p, LANES)]
                    g0, g1 = unpack(f_v[par, pl.ds(p, LANES)])
                    acc0 = acc0 + g0 * w
                    acc1 = acc1 + g1 * w
                pos = (s + iota) * (L * F) + (2 * lvl)
                plsc.store_scatter(out_v, [pos], acc0)
                plsc.store_scatter(out_v, [pos + 1], acc1)

            plsc.parallel_loop(0, NG, unroll=2)(work)

        def vmem_level(lvl):
            res, off = RES[lvl], OFFS[lvl]

            def work(g, lvl=lvl, res=res, off=off):
                s = g * LANES
                idx, w = corners(g, res, off, False)
                acc0 = jnp.zeros((LANES,), jnp.float32)
                acc1 = jnp.zeros((LANES,), jnp.float32)
                for k in range(4):
                    g0, g1 = unpack(plsc.load_gather(tab_v, [idx[k]]))
                    acc0 = acc0 + g0 * w[k]
                    acc1 = acc1 + g1 * w[k]
                pos = (s + iota) * (L * F) + (2 * lvl)
                plsc.store_scatter(out_v, [pos], acc0)
                plsc.store_scatter(out_v, [pos + 1], acc1)

            plsc.parallel_loop(0, NG, unroll=2)(work)

        pass_a(NLVL_VMEM, NLVL_VMEM % 2)
        cps = {NLVL_VMEM: fire(NLVL_VMEM, NLVL_VMEM % 2)}
        for lvl in range(NLVL_VMEM):
            vmem_level(lvl)
        for lvl in range(NLVL_VMEM, L):
            if lvl + 1 < L:
                par = (lvl + 1) % 2
                pass_a(lvl + 1, par)
                cps[lvl + 1] = fire(lvl + 1, par)
            cps.pop(lvl).wait()
            pass_b(lvl, lvl % 2)

        pltpu.sync_copy(out_v, out_hbm.at[pl.ds(pt0 * (L * F), C * L * F)])
        return 0

    lax.fori_loop(0, NCHUNK, chunk_body, 0)


@jax.jit
def kernel(uv, latents):
    u = uv[:, 0]
    v = uv[:, 1]
    packed = jax.lax.bitcast_convert_type(
        latents.astype(jnp.bfloat16), jnp.int32)
    mesh = plsc.VectorSubcoreMesh(
        core_axis_name="c", subcore_axis_name="s",
        num_cores=NC, num_subcores=NS)
    run = pl.kernel(
        _body,
        out_type=jax.ShapeDtypeStruct((N_POINTS * L * F,), jnp.float32),
        mesh=mesh,
        compiler_params=pltpu.CompilerParams(
            needs_layout_passes=False, use_tc_tiling_on_sc=False),
        scratch_types=[
            pltpu.VMEM((C,), jnp.float32),
            pltpu.VMEM((C,), jnp.float32),
            pltpu.VMEM((TAB_ROWS,), jnp.int32),
            pltpu.VMEM_SHARED((SP_ROWS,), jnp.int32),
            pltpu.VMEM((2, 4 * C), jnp.int32),
            pltpu.VMEM((2, 4 * C), jnp.float32),
            pltpu.VMEM((2, 4 * C), jnp.int32),
            pltpu.VMEM((C * L * F,), jnp.float32),
            pltpu.SemaphoreType.DMA((2,)),
        ],
    )
    return run(u, v, packed).reshape(N_POINTS, L * F)
